# Initial kernel scaffold; baseline (speedup 1.0000x reference)
import jax
import jax.numpy as jnp
from jax.experimental import pallas as pl


def kernel(x, edge_index, W, b, in_emb, out_emb):
    raise NotImplementedError("write your pallas kernel here")


# speedup vs baseline: 1.6846x; 1.6846x over previous
import functools

import jax
import jax.numpy as jnp
from jax import lax
from jax.experimental import pallas as pl
from jax.experimental.pallas import tpu as pltpu
from jax.experimental.pallas import tpu_sc as plsc

N_NODES = 50000
N_EDGES = 1600000
IN_DIM = 34
EMB_DIM = 256
MAX_DEG = 256

CHUNK = 128
GROUP = 8
N_CHUNKS = N_EDGES // CHUNK
N_GROUPS = N_CHUNKS // GROUP
TAIL_CHUNKS = N_CHUNKS - N_GROUPS * GROUP
N_SUBCORES = 16
N_ITERS = -(-N_GROUPS // N_SUBCORES)
N_PAD = 50176
SLICE = N_PAD // N_SUBCORES


def _sc_degree_body(e_ref, tail_ref, in_out, out_out, idx_win, ones_win,
                    idx_tail, zbuf, cnt):
    c = lax.axis_index("c")
    s = lax.axis_index("s")
    row = 1 - c

    zeros16 = jnp.zeros((16,), jnp.int32)
    ones16 = jnp.ones((16,), jnp.int32)

    @pl.loop(0, SLICE // 16)
    def _z(i):
        zbuf[pl.ds(i * 16, 16)] = zeros16

    @pl.loop(0, GROUP)
    def _o(i):
        @pl.loop(0, CHUNK // 16)
        def _oj(j):
            ones_win[i, pl.ds(j * 16, 16)] = ones16

    pltpu.sync_copy(zbuf, cnt.at[pl.ds(s * SLICE, SLICE)])
    plsc.subcore_barrier()

    @pl.loop(0, N_ITERS)
    def _w(i):
        g = i * N_SUBCORES + s

        @pl.when(g < N_GROUPS)
        def _do():
            base = pl.multiple_of(g * GROUP, GROUP)
            pltpu.sync_copy(e_ref.at[row, pl.ds(base, GROUP), :], idx_win)

            @pl.loop(0, GROUP)
            def _j(j):
                pltpu.sync_copy(ones_win.at[j], cnt.at[idx_win.at[j]],
                                add=True)

    @pl.when(s == 0)
    def _tail():
        pltpu.sync_copy(tail_ref.at[row], idx_tail)

        @pl.loop(0, TAIL_CHUNKS)
        def _tj(j):
            pltpu.sync_copy(ones_win.at[j], cnt.at[idx_tail.at[j]], add=True)

    plsc.subcore_barrier()
    pltpu.sync_copy(cnt.at[pl.ds(s * SLICE, SLICE)], zbuf)

    @pl.when(c == 0)
    def _w_in():
        pltpu.sync_copy(zbuf, in_out.at[pl.ds(s * SLICE, SLICE)])

    @pl.when(c == 1)
    def _w_out():
        pltpu.sync_copy(zbuf, out_out.at[pl.ds(s * SLICE, SLICE)])


_sc_degrees = functools.partial(
    pl.kernel,
    out_type=(jax.ShapeDtypeStruct((N_PAD,), jnp.int32),
              jax.ShapeDtypeStruct((N_PAD,), jnp.int32)),
    mesh=plsc.VectorSubcoreMesh(core_axis_name="c", subcore_axis_name="s"),
    scratch_types=[
        pltpu.VMEM((GROUP, CHUNK), jnp.int32),
        pltpu.VMEM((GROUP, CHUNK), jnp.int32),
        pltpu.VMEM((TAIL_CHUNKS, CHUNK), jnp.int32),
        pltpu.VMEM((SLICE,), jnp.int32),
        pltpu.VMEM_SHARED((N_PAD,), jnp.int32),
    ],
)(_sc_degree_body)


BLK = 1000


def _tc_fused_body(x_ref, wt_ref, b_ref, ind_ref, outd_ref, iemb_ref,
                   oemb_ref, o_ref):
    h = jnp.dot(x_ref[...], wt_ref[...], preferred_element_type=jnp.float32)
    iota = lax.broadcasted_iota(jnp.int32, (BLK, MAX_DEG), 1)
    ind = jnp.minimum(ind_ref[...], MAX_DEG - 1)
    outd = jnp.minimum(outd_ref[...], MAX_DEG - 1)
    oh_in = (ind == iota).astype(jnp.bfloat16)
    oh_out = (outd == iota).astype(jnp.bfloat16)
    h = h + jnp.dot(oh_in, iemb_ref[...], preferred_element_type=jnp.float32)
    h = h + jnp.dot(oh_out, oemb_ref[...], preferred_element_type=jnp.float32)
    o_ref[...] = h + b_ref[...]


def _tc_fused(x, w_t, b2, ind, outd, iemb, oemb):
    grid = (N_NODES // BLK,)
    return pl.pallas_call(
        _tc_fused_body,
        grid=grid,
        in_specs=[
            pl.BlockSpec((BLK, IN_DIM), lambda i: (i, 0)),
            pl.BlockSpec((IN_DIM, EMB_DIM), lambda i: (0, 0)),
            pl.BlockSpec((1, EMB_DIM), lambda i: (0, 0)),
            pl.BlockSpec((BLK, 1), lambda i: (i, 0)),
            pl.BlockSpec((BLK, 1), lambda i: (i, 0)),
            pl.BlockSpec((MAX_DEG, EMB_DIM), lambda i: (0, 0)),
            pl.BlockSpec((MAX_DEG, EMB_DIM), lambda i: (0, 0)),
        ],
        out_specs=pl.BlockSpec((BLK, EMB_DIM), lambda i: (i, 0)),
        out_shape=jax.ShapeDtypeStruct((N_NODES, EMB_DIM), jnp.float32),
    )(x, w_t, b2, ind, outd, iemb, oemb)


def kernel(x, edge_index, W, b, in_emb, out_emb):
    e3 = edge_index.reshape(2, N_CHUNKS, CHUNK)
    tail = e3[:, N_GROUPS * GROUP:, :]
    deg_in, deg_out = _sc_degrees(e3, tail)
    ind = deg_in[:N_NODES].reshape(N_NODES, 1)
    outd = deg_out[:N_NODES].reshape(N_NODES, 1)
    return _tc_fused(x, W.T, b.reshape(1, EMB_DIM), ind, outd,
                     in_emb.astype(jnp.bfloat16), out_emb.astype(jnp.bfloat16))


# speedup vs baseline: 3.0367x; 1.8025x over previous
import functools

import jax
import jax.numpy as jnp
from jax import lax
from jax.experimental import pallas as pl
from jax.experimental.pallas import tpu as pltpu
from jax.experimental.pallas import tpu_sc as plsc

N_NODES = 50000
N_EDGES = 1600000
IN_DIM = 34
EMB_DIM = 256
MAX_DEG = 256

CHUNK = 128
N_ROWS = 2 * N_EDGES // CHUNK
ROW_SPLIT = N_EDGES // CHUNK
GROUP = 8
N_GROUPS = 1562
BOUNDARY = 12496
N_SUBCORES = 16
N_ITERS = -(-N_GROUPS // N_SUBCORES)
N_PAIRS = N_ITERS // 2
N_PAD = 50176
SLICE = N_PAD // N_SUBCORES


def _sc_degree_body(e_ref, in_out, out_out, buf_a, buf_b, ones_row, zbuf,
                    cnt, sem_a, sem_b, sem_s):
    c = lax.axis_index("c")
    s = lax.axis_index("s")
    row0 = (1 - c) * 12504

    zeros16 = jnp.zeros((16,), jnp.int32)
    ones16 = jnp.ones((16,), jnp.int32)

    @pl.loop(0, SLICE // 16)
    def _z(i):
        zbuf[pl.ds(i * 16, 16)] = zeros16

    @pl.loop(0, CHUNK // 16)
    def _o(j):
        ones_row[pl.ds(j * 16, 16)] = ones16

    pltpu.sync_copy(zbuf, cnt.at[pl.ds(s * SLICE, SLICE)])
    plsc.subcore_barrier()

    def stage(buf, sem, g):
        base = pl.multiple_of(row0 + g * GROUP, GROUP)
        return pltpu.async_copy(e_ref.at[pl.ds(base, GROUP), :], buf, sem)

    def stage_wait(buf, sem, g):
        base = pl.multiple_of(row0 + g * GROUP, GROUP)
        pltpu.make_async_copy(e_ref.at[pl.ds(base, GROUP), :], buf, sem).wait()

    def fire(buf):
        for j in range(GROUP):
            pltpu.async_copy(ones_row, cnt.at[buf.at[j]], sem_s, add=True)

    def drain(buf):
        for j in range(GROUP):
            pltpu.make_async_copy(ones_row, cnt.at[buf.at[j]], sem_s).wait()

    stage(buf_a, sem_a, s)

    @pl.loop(0, N_PAIRS)
    def _pair(k):
        g_a = (2 * k) * N_SUBCORES + s
        g_b = (2 * k + 1) * N_SUBCORES + s
        b_ok = g_b < N_GROUPS

        @pl.when(b_ok)
        def _sb():
            stage(buf_b, sem_b, g_b)

        stage_wait(buf_a, sem_a, g_a)
        fire(buf_a)

        @pl.when(b_ok)
        def _fb():
            stage_wait(buf_b, sem_b, g_b)
            fire(buf_b)

        drain(buf_a)

        @pl.when(b_ok)
        def _db():
            drain(buf_b)

        @pl.when(k < N_PAIRS - 1)
        def _sa():
            stage(buf_a, sem_a, (2 * k + 2) * N_SUBCORES + s)

    @pl.when(s == 0)
    def _tail():
        pltpu.sync_copy(e_ref.at[pl.ds(BOUNDARY, GROUP), :], buf_a)
        first = (1 - c) * 4

        @pl.loop(0, 4)
        def _tj(j):
            pltpu.sync_copy(ones_row, cnt.at[buf_a.at[first + j]], add=True)

    plsc.subcore_barrier()
    pltpu.sync_copy(cnt.at[pl.ds(s * SLICE, SLICE)], zbuf)

    @pl.when(c == 0)
    def _w_in():
        pltpu.sync_copy(zbuf, in_out.at[pl.ds(s * SLICE, SLICE)])

    @pl.when(c == 1)
    def _w_out():
        pltpu.sync_copy(zbuf, out_out.at[pl.ds(s * SLICE, SLICE)])


_sc_degrees = functools.partial(
    pl.kernel,
    out_type=(jax.ShapeDtypeStruct((N_PAD,), jnp.int32),
              jax.ShapeDtypeStruct((N_PAD,), jnp.int32)),
    mesh=plsc.VectorSubcoreMesh(core_axis_name="c", subcore_axis_name="s"),
    scratch_types=[
        pltpu.VMEM((GROUP, CHUNK), jnp.int32),
        pltpu.VMEM((GROUP, CHUNK), jnp.int32),
        pltpu.VMEM((CHUNK,), jnp.int32),
        pltpu.VMEM((SLICE,), jnp.int32),
        pltpu.VMEM_SHARED((N_PAD,), jnp.int32),
        pltpu.SemaphoreType.DMA,
        pltpu.SemaphoreType.DMA,
        pltpu.SemaphoreType.DMA,
    ],
)(_sc_degree_body)


BLK = 1024


def _tc_fused_body(x_ref, wt_ref, b_ref, ind_ref, outd_ref, iemb_ref,
                   oemb_ref, o_ref):
    h = jnp.dot(x_ref[...], wt_ref[...], preferred_element_type=jnp.float32)
    iota = lax.broadcasted_iota(jnp.int32, (MAX_DEG, BLK), 0)
    ind = jnp.minimum(ind_ref[...], MAX_DEG - 1)
    outd = jnp.minimum(outd_ref[...], MAX_DEG - 1)
    oht_in = (ind == iota).astype(jnp.bfloat16)
    oht_out = (outd == iota).astype(jnp.bfloat16)
    dn = (((0,), (0,)), ((), ()))
    h = h + lax.dot_general(oht_in, iemb_ref[...], dn,
                            preferred_element_type=jnp.float32)
    h = h + lax.dot_general(oht_out, oemb_ref[...], dn,
                            preferred_element_type=jnp.float32)
    o_ref[...] = h + b_ref[...]


def _tc_fused(x, w_t, b2, ind, outd, iemb, oemb):
    grid = (N_PAD // BLK,)
    return pl.pallas_call(
        _tc_fused_body,
        grid=grid,
        in_specs=[
            pl.BlockSpec((BLK, IN_DIM), lambda i: (i, 0)),
            pl.BlockSpec((IN_DIM, EMB_DIM), lambda i: (0, 0)),
            pl.BlockSpec((1, EMB_DIM), lambda i: (0, 0)),
            pl.BlockSpec((BLK,), lambda i: (i,)),
            pl.BlockSpec((BLK,), lambda i: (i,)),
            pl.BlockSpec((MAX_DEG, EMB_DIM), lambda i: (0, 0)),
            pl.BlockSpec((MAX_DEG, EMB_DIM), lambda i: (0, 0)),
        ],
        out_specs=pl.BlockSpec((BLK, EMB_DIM), lambda i: (i, 0)),
        out_shape=jax.ShapeDtypeStruct((N_NODES, EMB_DIM), jnp.float32),
    )(x, w_t, b2, ind, outd, iemb, oemb)


def kernel(x, edge_index, W, b, in_emb, out_emb):
    e2 = edge_index.reshape(N_ROWS, CHUNK)
    deg_in, deg_out = _sc_degrees(e2)
    return _tc_fused(x, W.T, b.reshape(1, EMB_DIM), deg_in, deg_out,
                     in_emb.astype(jnp.bfloat16), out_emb.astype(jnp.bfloat16))
